# baseline (speedup 1.0000x reference)
import functools

import jax
import jax.numpy as jnp
from jax import lax
from jax.experimental import pallas as pl
from jax.experimental.pallas import tpu as pltpu
from jax.experimental.pallas import tpu_sc as plsc

_K = 10
_BETA = 0.15
_QB = 4
_ROWS = 3200


def _rerank_one(idx_row, ids_row, v_row, v_col, xq):
    m = xq.shape[0]
    sub = lax.broadcasted_iota(jnp.int32, (m, m), 0)
    lane = lax.broadcasted_iota(jnp.int32, (m, m), 1)

    sel_iota = lax.broadcasted_iota(jnp.int32, (m, _K), 0)
    mask = jnp.any(sel_iota == idx_row[:, :_K], axis=1, keepdims=True)
    x1 = jnp.max(jnp.where(mask, xq, -jnp.inf), axis=0, keepdims=True)

    x1_8 = jnp.broadcast_to(x1.astype(jnp.bfloat16), (8, x1.shape[1]))
    s_col = lax.dot_general(
        xq.astype(jnp.bfloat16), x1_8,
        (((1,), (1,)), ((), ())),
        preferred_element_type=jnp.float32,
    )[:, 0:1]

    g_row = jnp.sum(jnp.where(sub == idx_row, s_col, 0.0),
                    axis=0, keepdims=True)

    gt = jnp.where(v_row > v_col, 1, 0)
    eq = jnp.where((v_row == v_col) & (lane < sub), 1, 0)
    rank2_col = jnp.sum(gt + eq, axis=1, keepdims=True)
    sorted_row = jnp.sum(jnp.where(rank2_col == lane, v_col, 0.0),
                         axis=0, keepdims=True)

    rr_row = (sorted_row + g_row) * 0.5
    rr_col = jnp.sum(jnp.where(sub == lane, rr_row, 0.0),
                     axis=1, keepdims=True)

    r_gt = jnp.where(rr_col > rr_row, 1, 0)
    r_eq = jnp.where((rr_col == rr_row) & (sub < lane), 1, 0)
    rank3_row = jnp.sum(r_gt + r_eq, axis=0, keepdims=True)

    return jnp.sum(jnp.where(rank3_row == sub, ids_row, 0),
                   axis=1, keepdims=True)


def _body(idx_ref, ids_ref, vrow_ref, vcol_ref, x_ref, out_ref, head_ref):
    i = pl.program_id(0)
    ni = pl.num_programs(0)

    @pl.when(i == 0)
    def _():
        head_ref[...] = jnp.zeros_like(head_ref)

    qlane = lax.broadcasted_iota(jnp.int32, head_ref.shape, 1)
    acc = jnp.zeros(head_ref.shape, jnp.int32)
    for t in range(_QB):
        col = _rerank_one(idx_ref[t], ids_ref[t], vrow_ref[t], vcol_ref[t],
                          x_ref[t])
        acc = acc + jnp.where(qlane == _QB * i + t, col, 0)
    head_ref[...] += acc

    @pl.when(i == ni - 1)
    def _():
        out_ref[...] = head_ref[...]


def _join_body(out0_ref, head_ref, out_ref):
    out_ref[...] = head_ref[...]


@jax.jit
def kernel(ranks, rerank_dba_final, res_top1000_dba, ranks_trans_1000_pre, x_dba):
    n, q = ranks.shape
    _, m, d = x_dba.shape

    idx3 = ranks_trans_1000_pre.reshape(q, 1, m)
    ids3 = rerank_dba_final.reshape(q, 1, m)
    vrow3 = res_top1000_dba.reshape(q, 1, m)
    vcol3 = res_top1000_dba.reshape(q, m, 1)

    head = pl.pallas_call(
        _body,
        grid=(q // _QB,),
        in_specs=[
            pl.BlockSpec((_QB, 1, m), lambda i: (i, 0, 0)),
            pl.BlockSpec((_QB, 1, m), lambda i: (i, 0, 0)),
            pl.BlockSpec((_QB, 1, m), lambda i: (i, 0, 0)),
            pl.BlockSpec((_QB, m, 1), lambda i: (i, 0, 0)),
            pl.BlockSpec((_QB, m, d), lambda i: (i, 0, 0)),
        ],
        out_specs=pl.BlockSpec((m, q), lambda i: (0, 0)),
        out_shape=jax.ShapeDtypeStruct((m, q), jnp.int32),
        scratch_shapes=[pltpu.VMEM((m, q), jnp.int32)],
    )(idx3, ids3, vrow3, vcol3, x_dba)

    mesh = plsc.ScalarSubcoreMesh(axis_name="core", num_cores=2)

    @functools.partial(
        pl.kernel,
        out_type=jax.ShapeDtypeStruct((n, q), jnp.int32),
        mesh=mesh,
        scratch_types=[pltpu.SemaphoreType.DMA],
    )
    def _sc_tail_copy(ranks_hbm, out_hbm, sem):
        c = lax.axis_index("core")
        half = (n - m) // 2
        start = m + c * half
        pltpu.async_copy(
            ranks_hbm.at[pl.ds(start, half)],
            out_hbm.at[pl.ds(start, half)],
            sem,
        ).wait()

    out0 = _sc_tail_copy(ranks)

    out = pl.pallas_call(
        _join_body,
        grid=(1,),
        in_specs=[
            pl.BlockSpec((m, q), lambda i: (0, 0)),
            pl.BlockSpec((m, q), lambda i: (0, 0)),
        ],
        out_specs=pl.BlockSpec((m, q), lambda i: (0, 0)),
        out_shape=jax.ShapeDtypeStruct((n, q), jnp.int32),
        input_output_aliases={0: 0},
    )(out0, head)
    return out


# speedup vs baseline: 7.9538x; 7.9538x over previous
import jax
import jax.numpy as jnp
from jax import lax
from jax.experimental import pallas as pl
from jax.experimental.pallas import tpu as pltpu

_K = 10
_BETA = 0.15
_QB = 8
_ROWS = 6400


def _rerank_one(idx_row, ids_row, v_row, v_col, xq):
    m = xq.shape[0]
    sub = lax.broadcasted_iota(jnp.int32, (m, m), 0)
    lane = lax.broadcasted_iota(jnp.int32, (m, m), 1)

    sel_iota = lax.broadcasted_iota(jnp.int32, (m, _K), 0)
    mask = jnp.any(sel_iota == idx_row[:, :_K], axis=1, keepdims=True)
    x1 = jnp.max(jnp.where(mask, xq, -jnp.inf), axis=0, keepdims=True)

    x1_8 = jnp.broadcast_to(x1.astype(jnp.bfloat16), (8, x1.shape[1]))
    s_col = lax.dot_general(
        xq.astype(jnp.bfloat16), x1_8,
        (((1,), (1,)), ((), ())),
        preferred_element_type=jnp.float32,
    )[:, 0:1]

    g_row = jnp.sum(jnp.where(sub == idx_row, s_col, 0.0),
                    axis=0, keepdims=True)

    gt = jnp.where(v_row > v_col, 1, 0)
    eq = jnp.where((v_row == v_col) & (lane < sub), 1, 0)
    rank2_col = jnp.sum(gt + eq, axis=1, keepdims=True)
    sorted_row = jnp.sum(jnp.where(rank2_col == lane, v_col, 0.0),
                         axis=0, keepdims=True)

    rr_row = (sorted_row + g_row) * 0.5
    rr_col = jnp.sum(jnp.where(sub == lane, rr_row, 0.0),
                     axis=1, keepdims=True)

    r_gt = jnp.where(rr_col > rr_row, 1, 0)
    r_eq = jnp.where((rr_col == rr_row) & (sub < lane), 1, 0)
    rank3_row = jnp.sum(r_gt + r_eq, axis=0, keepdims=True)

    return jnp.sum(jnp.where(rank3_row == sub, ids_row, 0),
                   axis=1, keepdims=True)


def _body(idx_ref, ids_ref, vrow_ref, vcol_ref, x_ref, ranks_ref,
          out_ref, head_ref):
    i = pl.program_id(0)
    ni = pl.num_programs(0)
    m = x_ref.shape[1]

    @pl.when(i == 0)
    def _():
        head_ref[...] = jnp.zeros_like(head_ref)

    qlane = lax.broadcasted_iota(jnp.int32, head_ref.shape, 1)
    acc = jnp.zeros(head_ref.shape, jnp.int32)
    for t in range(_QB):
        col = _rerank_one(idx_ref[t], ids_ref[t], vrow_ref[t], vcol_ref[t],
                          x_ref[t])
        acc = acc + jnp.where(qlane == _QB * i + t, col, 0)
    head_ref[...] += acc

    out_ref[...] = ranks_ref[...]

    @pl.when(i == ni - 1)
    def _():
        out_ref[0:m, :] = head_ref[...]


def _omap(i):
    return jnp.where(i == 15, 0, jnp.minimum(i + 1, 15))


@jax.jit
def kernel(ranks, rerank_dba_final, res_top1000_dba, ranks_trans_1000_pre, x_dba):
    n, q = ranks.shape
    _, m, d = x_dba.shape

    idx3 = ranks_trans_1000_pre.reshape(q, 1, m)
    ids3 = rerank_dba_final.reshape(q, 1, m)
    vrow3 = res_top1000_dba.reshape(q, 1, m)
    vcol3 = res_top1000_dba.reshape(q, m, 1)

    out = pl.pallas_call(
        _body,
        grid=(q // _QB,),
        in_specs=[
            pl.BlockSpec((_QB, 1, m), lambda i: (i, 0, 0)),
            pl.BlockSpec((_QB, 1, m), lambda i: (i, 0, 0)),
            pl.BlockSpec((_QB, 1, m), lambda i: (i, 0, 0)),
            pl.BlockSpec((_QB, m, 1), lambda i: (i, 0, 0)),
            pl.BlockSpec((_QB, m, d), lambda i: (i, 0, 0)),
            pl.BlockSpec((_ROWS, q), lambda i: (_omap(i), 0)),
        ],
        out_specs=pl.BlockSpec((_ROWS, q), lambda i: (_omap(i), 0)),
        out_shape=jax.ShapeDtypeStruct((n, q), jnp.int32),
        scratch_shapes=[pltpu.VMEM((m, q), jnp.int32)],
    )(idx3, ids3, vrow3, vcol3, x_dba, ranks)
    return out


# speedup vs baseline: 9.2674x; 1.1652x over previous
import jax
import jax.numpy as jnp
from jax import lax
from jax.experimental import pallas as pl
from jax.experimental.pallas import tpu as pltpu

_K = 10
_BETA = 0.15
_QB = 4
_ROWS = 3200


def _rerank_one(idx_row, ids_row, v_row, v_col, xq):
    m = xq.shape[0]
    sub = lax.broadcasted_iota(jnp.int32, (m, m), 0)
    lane = lax.broadcasted_iota(jnp.int32, (m, m), 1)

    sel_iota = lax.broadcasted_iota(jnp.int32, (m, _K), 0)
    mask = jnp.any(sel_iota == idx_row[:, :_K], axis=1, keepdims=True)
    x1 = jnp.max(jnp.where(mask, xq, -jnp.inf), axis=0, keepdims=True)

    x1_8 = jnp.broadcast_to(x1.astype(jnp.bfloat16), (8, x1.shape[1]))
    s_col = lax.dot_general(
        xq.astype(jnp.bfloat16), x1_8,
        (((1,), (1,)), ((), ())),
        preferred_element_type=jnp.float32,
    )[:, 0:1]

    g_row = jnp.sum(jnp.where(sub == idx_row, s_col, 0.0),
                    axis=0, keepdims=True)

    gt = jnp.where(v_row > v_col, 1, 0)
    eq = jnp.where((v_row == v_col) & (lane < sub), 1, 0)
    rank2_col = jnp.sum(gt + eq, axis=1, keepdims=True)
    sorted_row = jnp.sum(jnp.where(rank2_col == lane, v_col, 0.0),
                         axis=0, keepdims=True)

    rr_row = (sorted_row + g_row) * 0.5
    rr_col = jnp.sum(jnp.where(sub == lane, rr_row, 0.0),
                     axis=1, keepdims=True)

    r_gt = jnp.where(rr_col > rr_row, 1, 0)
    r_eq = jnp.where((rr_col == rr_row) & (sub < lane), 1, 0)
    rank3_row = jnp.sum(r_gt + r_eq, axis=0, keepdims=True)

    return jnp.sum(jnp.where(rank3_row == sub, ids_row, 0),
                   axis=1, keepdims=True)


def _body(idx_ref, ids_ref, vrow_ref, vcol_ref, x_ref, ranks_ref,
          out_ref, head_ref):
    i = pl.program_id(0)
    ni = pl.num_programs(0)
    m = x_ref.shape[1]

    @pl.when(i == 0)
    def _():
        head_ref[...] = jnp.zeros_like(head_ref)

    qlane = lax.broadcasted_iota(jnp.int32, head_ref.shape, 1)
    acc = jnp.zeros(head_ref.shape, jnp.int32)
    for t in range(_QB):
        col = _rerank_one(idx_ref[t], ids_ref[t], vrow_ref[t], vcol_ref[t],
                          x_ref[t])
        acc = acc + jnp.where(qlane == _QB * i + t, col, 0)
    head_ref[...] += acc

    out_ref[...] = ranks_ref[...]

    @pl.when(i == ni - 1)
    def _():
        out_ref[0:m, :] = head_ref[...]


def _omap(i):
    return jnp.where(i == 31, 0, jnp.minimum(i + 1, 31))


@jax.jit
def kernel(ranks, rerank_dba_final, res_top1000_dba, ranks_trans_1000_pre, x_dba):
    n, q = ranks.shape
    _, m, d = x_dba.shape

    idx3 = ranks_trans_1000_pre.reshape(q, 1, m)
    ids3 = rerank_dba_final.reshape(q, 1, m)
    vrow3 = res_top1000_dba.reshape(q, 1, m)
    vcol3 = res_top1000_dba.reshape(q, m, 1)

    out = pl.pallas_call(
        _body,
        grid=(q // _QB,),
        in_specs=[
            pl.BlockSpec((_QB, 1, m), lambda i: (i, 0, 0)),
            pl.BlockSpec((_QB, 1, m), lambda i: (i, 0, 0)),
            pl.BlockSpec((_QB, 1, m), lambda i: (i, 0, 0)),
            pl.BlockSpec((_QB, m, 1), lambda i: (i, 0, 0)),
            pl.BlockSpec((_QB, m, d), lambda i: (i, 0, 0)),
            pl.BlockSpec((_ROWS, q), lambda i: (_omap(i), 0)),
        ],
        out_specs=pl.BlockSpec((_ROWS, q), lambda i: (_omap(i), 0)),
        out_shape=jax.ShapeDtypeStruct((n, q), jnp.int32),
        scratch_shapes=[pltpu.VMEM((m, q), jnp.int32)],
    )(idx3, ids3, vrow3, vcol3, x_dba, ranks)
    return out
